# Initial kernel scaffold; baseline (speedup 1.0000x reference)
import jax
import jax.numpy as jnp
from jax.experimental import pallas as pl


def kernel(activations, attention_weights, neighbor_weights, neighbor_indices):
    raise NotImplementedError("write your pallas kernel here")


# speedup vs baseline: 4.9120x; 4.9120x over previous
import functools

import jax
import jax.numpy as jnp
from jax import lax
from jax.experimental import pallas as pl
from jax.experimental.pallas import tpu as pltpu
from jax.experimental.pallas import tpu_sc as plsc

N = 10000
K = 64
B = 64
ALPHA = 0.1
DELTA = 0.05
GAMMA = 0.5
EPSILON = 1e-06

NC = 2
NS = 16
NW = NC * NS
NP = 10240
RPW = NP // NW


def _tc_prep_body(act_ref, attw_ref, nw_ref, a_ref, w_ref):
    a_ref[...] = (1.0 - DELTA) * act_ref[...] + GAMMA * attw_ref[...]
    x = nw_ref[...]
    m = jnp.max(x, axis=-1, keepdims=True)
    e = jnp.exp(x - m)
    w_ref[...] = e / jnp.sum(e, axis=-1, keepdims=True)


def _tc_prep(act, attw, nw):
    return pl.pallas_call(
        _tc_prep_body,
        out_shape=(
            jax.ShapeDtypeStruct((B, N), jnp.float32),
            jax.ShapeDtypeStruct((N, K), jnp.float32),
        ),
    )(act, attw, nw)


def _sc_spread_body(a_hbm, w_hbm, idx_hbm, out_hbm,
                    idx_v, w_v, a_v, out_v, gbuf, sem):
    wid = lax.axis_index("s") * NC + lax.axis_index("c")
    base = wid * RPW
    pltpu.sync_copy(idx_hbm.at[pl.ds(base, RPW)], idx_v)
    pltpu.sync_copy(w_hbm.at[pl.ds(base, RPW)], w_v)
    pltpu.sync_copy(a_hbm.at[pl.ds(base, RPW)], a_v)

    def body(i, _):
        pltpu.async_copy(a_hbm.at[idx_v.at[i]], gbuf, sem).wait()
        accs = [jnp.zeros((16,), jnp.float32) for _ in range(4)]
        for kc in range(K // 16):
            wvec = w_v[i, pl.ds(kc * 16, 16)]
            for kk in range(16):
                k = kc * 16 + kk
                wk = wvec[kk]
                for c in range(4):
                    accs[c] = accs[c] + wk * gbuf[k, pl.ds(c * 16, 16)]
        for c in range(4):
            sl = pl.ds(c * 16, 16)
            val = a_v[i, sl] + ALPHA * accs[c]
            out_v[i, sl] = jnp.clip(val, EPSILON, 1.0)
        return 0

    lax.fori_loop(0, RPW, body, 0)
    pltpu.sync_copy(out_v, out_hbm.at[pl.ds(base, RPW)])


@functools.cache
def _sc_spread():
    return pl.kernel(
        _sc_spread_body,
        out_type=jax.ShapeDtypeStruct((NP, B), jnp.float32),
        mesh=plsc.VectorSubcoreMesh(core_axis_name="c", subcore_axis_name="s",
                                    num_cores=NC, num_subcores=NS),
        scratch_types=[
            pltpu.VMEM((RPW, K), jnp.int32),
            pltpu.VMEM((RPW, K), jnp.float32),
            pltpu.VMEM((RPW, B), jnp.float32),
            pltpu.VMEM((RPW, B), jnp.float32),
            pltpu.VMEM((K, B), jnp.float32),
            pltpu.SemaphoreType.DMA,
        ],
        compiler_params=pltpu.CompilerParams(use_tc_tiling_on_sc=False),
    )


def kernel(activations, attention_weights, neighbor_weights, neighbor_indices):
    a, w = _tc_prep(activations, attention_weights, neighbor_weights)
    a_t = jnp.pad(a.T, ((0, NP - N), (0, 0)))
    w_p = jnp.pad(w, ((0, NP - N), (0, 0)))
    idx_p = jnp.pad(neighbor_indices.astype(jnp.int32), ((0, NP - N), (0, 0)))
    out_t = _sc_spread()(a_t, w_p, idx_p)
    return out_t[:N].T


# speedup vs baseline: 6.4461x; 1.3123x over previous
import functools

import jax
import jax.numpy as jnp
from jax import lax
from jax.experimental import pallas as pl
from jax.experimental.pallas import tpu as pltpu
from jax.experimental.pallas import tpu_sc as plsc

N = 10000
K = 64
B = 64
ALPHA = 0.1
DELTA = 0.05
GAMMA = 0.5
EPSILON = 1e-06

NC = 2
NS = 16
NW = NC * NS
NP = 10240
RPW = NP // NW


def _tc_prep_body(act_ref, attw_ref, nw_ref, a_ref, w_ref):
    a_ref[...] = (1.0 - DELTA) * act_ref[...] + GAMMA * attw_ref[...]
    x = nw_ref[...]
    m = jnp.max(x, axis=-1, keepdims=True)
    e = jnp.exp(x - m)
    w_ref[...] = e / jnp.sum(e, axis=-1, keepdims=True)


def _tc_prep(act, attw, nw):
    return pl.pallas_call(
        _tc_prep_body,
        out_shape=(
            jax.ShapeDtypeStruct((B, N), jnp.float32),
            jax.ShapeDtypeStruct((N, K), jnp.float32),
        ),
    )(act, attw, nw)


NBUF = 2


def _sc_spread_body(a_hbm, w_hbm, idx_hbm, out_hbm,
                    idx_v, w_v, a_v, out_v, gbuf, sem0, sem1):
    wid = lax.axis_index("s") * NC + lax.axis_index("c")
    base = wid * RPW
    pltpu.sync_copy(idx_hbm.at[pl.ds(base, RPW)], idx_v)
    pltpu.sync_copy(w_hbm.at[pl.ds(base, RPW)], w_v)
    pltpu.sync_copy(a_hbm.at[pl.ds(base, RPW)], a_v)

    sems = (sem0, sem1)

    def fire(i, b):
        pltpu.async_copy(a_hbm.at[idx_v.at[i]], gbuf.at[b], sems[b])

    def wait(i, b):
        pltpu.make_async_copy(a_hbm.at[idx_v.at[i]], gbuf.at[b], sems[b]).wait()

    def compute(i, b):
        accs = [jnp.zeros((16,), jnp.float32) for _ in range(4)]
        for kc in range(K // 16):
            wvec = w_v[i, pl.ds(kc * 16, 16)]
            for kk in range(16):
                k = kc * 16 + kk
                wk = wvec[kk]
                for c in range(4):
                    accs[c] = accs[c] + wk * gbuf[b, k, pl.ds(c * 16, 16)]
        for c in range(4):
            sl = pl.ds(c * 16, 16)
            val = a_v[i, sl] + ALPHA * accs[c]
            out_v[i, sl] = jnp.clip(val, EPSILON, 1.0)

    for b in range(NBUF):
        fire(b, b)

    def body(j, _):
        for b in range(NBUF):
            i = j * NBUF + b
            wait(i, b)
            compute(i, b)

            @pl.when(i + NBUF < RPW)
            def _():
                fire(i + NBUF, b)
        return 0

    lax.fori_loop(0, RPW // NBUF, body, 0)
    pltpu.sync_copy(out_v, out_hbm.at[pl.ds(base, RPW)])


@functools.cache
def _sc_spread():
    return pl.kernel(
        _sc_spread_body,
        out_type=jax.ShapeDtypeStruct((NP, B), jnp.float32),
        mesh=plsc.VectorSubcoreMesh(core_axis_name="c", subcore_axis_name="s",
                                    num_cores=NC, num_subcores=NS),
        scratch_types=[
            pltpu.VMEM((RPW, K), jnp.int32),
            pltpu.VMEM((RPW, K), jnp.float32),
            pltpu.VMEM((RPW, B), jnp.float32),
            pltpu.VMEM((RPW, B), jnp.float32),
            pltpu.VMEM((NBUF, K, B), jnp.float32),
            pltpu.SemaphoreType.DMA,
            pltpu.SemaphoreType.DMA,
        ],
        compiler_params=pltpu.CompilerParams(use_tc_tiling_on_sc=False),
    )


def kernel(activations, attention_weights, neighbor_weights, neighbor_indices):
    a, w = _tc_prep(activations, attention_weights, neighbor_weights)
    a_t = jnp.pad(a.T, ((0, NP - N), (0, 0)))
    w_p = jnp.pad(w, ((0, NP - N), (0, 0)))
    idx_p = jnp.pad(neighbor_indices.astype(jnp.int32), ((0, NP - N), (0, 0)))
    out_t = _sc_spread()(a_t, w_p, idx_p)
    return out_t[:N].T
